# Initial kernel scaffold; baseline (speedup 1.0000x reference)
import jax
import jax.numpy as jnp
from jax.experimental import pallas as pl


def kernel(x, table, W, b):
    raise NotImplementedError("write your pallas kernel here")


# speedup vs baseline: 1.5846x; 1.5846x over previous
import functools

import jax
import jax.numpy as jnp
from jax import lax
from jax.experimental import pallas as pl
from jax.experimental.pallas import tpu as pltpu
from jax.experimental.pallas import tpu_sc as plsc

_B, _L, _EMB = 4096, 100, 768
_PROMPT = 100
_N = _B * _L
_VPAD = 128
_NC, _NS = 2, 16
_NW = _NC * _NS
_BPW = _N // _NW
_C = 80
_NSTEPS = _BPW // _C


def _y_body(t_ref, w_ref, b_ref, y_ref):
    h = lax.dot_general(t_ref[...], w_ref[...],
                        (((1,), (1,)), ((), ())),
                        preferred_element_type=jnp.float32)
    h = h + b_ref[...]
    y_ref[...] = 0.5 * h * (1.0 + lax.erf(h * 0.7071067811865476))


def _fold_table(table, W, b):
    tp = jnp.zeros((_VPAD, _EMB), jnp.float32).at[:_PROMPT, :].set(table)
    return pl.pallas_call(
        _y_body,
        out_shape=jax.ShapeDtypeStruct((_VPAD, _EMB), jnp.float32),
    )(tp, W, b.reshape(1, _EMB))


def _gather_body(y_hbm, idx_hbm, out_hbm, idx_v, rows_v, sem):
    wid = lax.axis_index("s") * _NC + lax.axis_index("c")
    base = wid * _BPW

    @pl.loop(0, _NSTEPS)
    def _step(i):
        start = base + i * _C
        pltpu.sync_copy(idx_hbm.at[pl.ds(start, _C)], idx_v)
        pltpu.async_copy(y_hbm.at[idx_v], rows_v, sem).wait()
        pltpu.sync_copy(rows_v, out_hbm.at[pl.ds(start, _C)])


_gather = functools.partial(
    pl.kernel,
    out_type=jax.ShapeDtypeStruct((_N, _EMB), jnp.float32),
    mesh=plsc.VectorSubcoreMesh(core_axis_name="c", subcore_axis_name="s"),
    scratch_types=[
        pltpu.VMEM((_C,), jnp.int32),
        pltpu.VMEM((_C, _EMB), jnp.float32),
        pltpu.SemaphoreType.DMA,
    ],
)(_gather_body)


def kernel(x, table, W, b):
    y = _fold_table(table, W, b)
    out = _gather(y, x.reshape(_N))
    return out.reshape(_B, _L, _EMB)


# speedup vs baseline: 1.5869x; 1.0014x over previous
import functools

import jax
import jax.numpy as jnp
from jax import lax
from jax.experimental import pallas as pl
from jax.experimental.pallas import tpu as pltpu
from jax.experimental.pallas import tpu_sc as plsc

_B, _L, _EMB = 4096, 100, 768
_PROMPT = 100
_N = _B * _L
_VPAD = 128
_NC, _NS = 2, 16
_NW = _NC * _NS
_BPW = _N // _NW
_C = 64
_NSTEPS = _BPW // _C


def _y_body(t_ref, w_ref, b_ref, y_ref):
    h = lax.dot_general(t_ref[...], w_ref[...],
                        (((1,), (1,)), ((), ())),
                        preferred_element_type=jnp.float32)
    h = h + b_ref[...]
    y_ref[...] = 0.5 * h * (1.0 + lax.erf(h * 0.7071067811865476))


def _fold_table(table, W, b):
    tp = jnp.zeros((_VPAD, _EMB), jnp.float32).at[:_PROMPT, :].set(table)
    return pl.pallas_call(
        _y_body,
        out_shape=jax.ShapeDtypeStruct((_VPAD, _EMB), jnp.float32),
    )(tp, W, b.reshape(1, _EMB))


def _gather_body(y_hbm, idx_hbm, out_hbm,
                 idx_v, rows0, rows1, gsem0, gsem1, wsem0, wsem1):
    wid = lax.axis_index("s") * _NC + lax.axis_index("c")
    base = wid * _BPW

    pltpu.sync_copy(idx_hbm.at[pl.ds(base, _BPW)], idx_v)

    def _gather(step, rows, sem):
        return pltpu.async_copy(
            y_hbm.at[idx_v.at[pl.ds(step * _C, _C)]], rows, sem)

    def _wait_gather(step, rows, sem):
        pltpu.make_async_copy(
            y_hbm.at[idx_v.at[pl.ds(step * _C, _C)]], rows, sem).wait()

    def _write(step, rows, sem):
        return pltpu.async_copy(
            rows, out_hbm.at[pl.ds(base + step * _C, _C)], sem)

    def _wait_write(step, rows, sem):
        pltpu.make_async_copy(
            rows, out_hbm.at[pl.ds(base + step * _C, _C)], sem).wait()

    _gather(0, rows0, gsem0)

    @pl.loop(0, _NSTEPS, step=2)
    def _pair(s):
        _wait_gather(s, rows0, gsem0)
        _write(s, rows0, wsem0)

        @pl.when(s > 0)
        def _():
            _wait_write(s - 1, rows1, wsem1)

        _gather(s + 1, rows1, gsem1)

        _wait_gather(s + 1, rows1, gsem1)
        _write(s + 1, rows1, wsem1)
        _wait_write(s, rows0, wsem0)

        @pl.when(s + 2 < _NSTEPS)
        def _():
            _gather(s + 2, rows0, gsem0)

    _wait_write(_NSTEPS - 1, rows1, wsem1)


_gather_call = functools.partial(
    pl.kernel,
    out_type=jax.ShapeDtypeStruct((_N, _EMB), jnp.float32),
    mesh=plsc.VectorSubcoreMesh(core_axis_name="c", subcore_axis_name="s"),
    scratch_types=[
        pltpu.VMEM((_BPW,), jnp.int32),
        pltpu.VMEM((_C, _EMB), jnp.float32),
        pltpu.VMEM((_C, _EMB), jnp.float32),
        pltpu.SemaphoreType.DMA,
        pltpu.SemaphoreType.DMA,
        pltpu.SemaphoreType.DMA,
        pltpu.SemaphoreType.DMA,
    ],
)(_gather_body)


def kernel(x, table, W, b):
    y = _fold_table(table, W, b)
    out = _gather_call(y, x.reshape(_N))
    return out.reshape(_B, _L, _EMB)
